# baseline (speedup 1.0000x reference)
import jax
import jax.numpy as jnp
from jax.experimental import pallas as pl
from jax.experimental.pallas import tpu as pltpu

SPLIT_INTO = 7
PERM = (2, 4, 3, 6, 5, 0, 1)
LANES = 128
NBUF = 3
AHEAD = 2
SLAB = 64


def kernel(x):
    b, t, f = x.shape
    rows = t * f // LANES
    crows = rows // SPLIT_INTO
    nslab = b // SLAB
    n = SPLIT_INTO * nslab

    def body(x_hbm, out_hbm, buf, sem_in, sem_out):
        def start_in(j):
            c, s = divmod(j, nslab)
            return pltpu.make_async_copy(
                x_hbm.at[pl.ds(s * SLAB, SLAB), pl.ds(PERM[c] * crows, crows), :],
                buf.at[j % NBUF],
                sem_in.at[j % NBUF],
            )

        def start_out(j):
            c, s = divmod(j, nslab)
            return pltpu.make_async_copy(
                buf.at[j % NBUF],
                out_hbm.at[pl.ds(s * SLAB, SLAB), pl.ds(c * crows, crows), :],
                sem_out.at[j % NBUF],
            )

        ins, outs = {}, {}
        for j in range(AHEAD):
            ins[j] = start_in(j)
            ins[j].start()
        for j in range(n):
            k = j + AHEAD
            if k < n:
                if k >= NBUF:
                    outs[k - NBUF].wait()
                ins[k] = start_in(k)
                ins[k].start()
            ins[j].wait()
            outs[j] = start_out(j)
            outs[j].start()
        for j in range(n - NBUF, n):
            outs[j].wait()

    xv = x.reshape(b, rows, LANES)
    out = pl.pallas_call(
        body,
        out_shape=jax.ShapeDtypeStruct((b, rows, LANES), jnp.float32),
        in_specs=[pl.BlockSpec(memory_space=pl.ANY)],
        out_specs=pl.BlockSpec(memory_space=pl.ANY),
        scratch_shapes=[
            pltpu.VMEM((NBUF, SLAB, crows, LANES), jnp.float32),
            pltpu.SemaphoreType.DMA((NBUF,)),
            pltpu.SemaphoreType.DMA((NBUF,)),
        ],
    )(xv)
    return out.reshape(b, t, f)
